# baseline (speedup 1.0000x reference)
import jax
import jax.numpy as jnp
from jax import lax
from jax.experimental import pallas as pl
from jax.experimental.pallas import tpu as pltpu
from jax.experimental.pallas import tpu_sc as plsc

VOCAB_SIZE = 1000000
EMBED_DIM = 32

NUM_CORES = 2
NUM_SUBCORES = 16
NUM_WORKERS = NUM_CORES * NUM_SUBCORES

NSEQ = 4096
SEQ_LEN = 200
B_TOTAL = NSEQ * SEQ_LEN
LANES = 128
ROWS_PER_LINE = LANES // EMBED_DIM
N_LINES = VOCAB_SIZE // ROWS_PER_LINE
OUT_LINES = B_TOTAL // ROWS_PER_LINE

B_PER_W = B_TOTAL // NUM_WORKERS
CHUNK = 128
N_CHUNKS = B_PER_W // CHUNK
OUT_LINES_PER_CHUNK = CHUNK // ROWS_PER_LINE
N_SLAB = 4
GROUPS = CHUNK // 16


def _gather_body(gidx_hbm, off_hbm, table_hbm, out_hbm,
                 gidx_v, off_v, s0, s1, s2, s3, o0, o1,
                 g0, g1, g2, g3, w0, w1):
    slabs = [s0, s1, s2, s3]
    gsems = [g0, g1, g2, g3]
    ostages = [o0, o1]
    osems = [w0, w1]

    wid = lax.axis_index("s") * NUM_CORES + lax.axis_index("c")
    pltpu.sync_copy(gidx_hbm.at[pl.ds(wid * N_CHUNKS, N_CHUNKS)], gidx_v)
    pltpu.sync_copy(off_hbm.at[pl.ds(wid * N_CHUNKS, N_CHUNKS)], off_v)
    out_base = wid * (OUT_LINES // NUM_WORKERS)

    iota = lax.iota(jnp.int32, 16)
    row_dst_base = iota >> 2
    col_dst_base = (iota & 3) * 32

    def fire_gather(c, b):
        return pltpu.async_copy(
            table_hbm.at[gidx_v.at[c]], slabs[b], gsems[b]
        )

    def drain_gather(b):
        pltpu.make_async_copy(
            table_hbm.at[pl.ds(0, CHUNK)], slabs[b], gsems[b]
        ).wait()

    def fire_store(c, o):
        return pltpu.async_copy(
            ostages[o],
            out_hbm.at[pl.ds(out_base + c * OUT_LINES_PER_CHUNK,
                             OUT_LINES_PER_CHUNK)],
            osems[o],
        )

    def drain_store(o):
        pltpu.make_async_copy(
            out_hbm.at[pl.ds(0, OUT_LINES_PER_CHUNK)], ostages[o], osems[o]
        ).wait()

    def extract(c, b, o):
        slab = slabs[b]
        ostage = ostages[o]
        offs_g = [off_v[c, pl.ds(16 * g, 16)] for g in range(GROUPS)]

        def fbody(f, carry):
            for g in range(GROUPS):
                v = plsc.load_gather(slab, [iota + 16 * g, offs_g[g] + f])
                plsc.store_scatter(
                    ostage, [row_dst_base + 4 * g, col_dst_base + f], v
                )
            return carry

        lax.fori_loop(0, EMBED_DIM, fbody, 0)

    def step(s, drain_st, fire_g):
        b = s % N_SLAB
        o = s % 2
        drain_gather(b)
        if drain_st:
            drain_store(o)
        extract(s, b, o)
        fire_store(s, o)
        if fire_g:
            fire_gather(s + (N_SLAB - 1), (s + (N_SLAB - 1)) % N_SLAB)

    for c in range(N_SLAB - 1):
        fire_gather(c, c)
    step(0, False, True)
    step(1, False, True)
    step(2, True, True)
    step(3, True, True)

    def main_body(i, carry):
        s4 = i * 4
        for off4 in range(4):
            s = s4 + off4
            b = off4 % N_SLAB
            o = off4 % 2
            drain_gather(b)
            drain_store(o)
            extract(s, b, o)
            fire_store(s, o)
            fire_gather(s + (N_SLAB - 1), (b + N_SLAB - 1) % N_SLAB)
        return carry

    lax.fori_loop(1, (N_CHUNKS - 4) // 4, main_body, 0)

    step(196, True, True)
    step(197, True, False)
    step(198, True, False)
    step(199, True, False)
    drain_store(0)
    drain_store(1)


@jax.jit
def _embed_gather(word_sequences, table):
    idx = word_sequences.astype(jnp.int32)
    gidx = (idx >> 2).reshape(B_TOTAL // LANES, LANES)
    offs = ((idx & 3) << 5).reshape(B_TOTAL // LANES, LANES)
    table4 = table.reshape(N_LINES, LANES)
    mesh = plsc.VectorSubcoreMesh(
        core_axis_name="c",
        subcore_axis_name="s",
        num_cores=NUM_CORES,
        num_subcores=NUM_SUBCORES,
    )
    out = pl.kernel(
        _gather_body,
        out_type=jax.ShapeDtypeStruct((OUT_LINES, LANES), jnp.float32),
        mesh=mesh,
        scratch_types=(
            [pltpu.VMEM((N_CHUNKS, LANES), jnp.int32)] * 2
            + [pltpu.VMEM((CHUNK, LANES), jnp.float32) for _ in range(N_SLAB)]
            + [pltpu.VMEM((OUT_LINES_PER_CHUNK, LANES), jnp.float32)] * 2
            + [pltpu.SemaphoreType.DMA for _ in range(N_SLAB + 2)]
        ),
        compiler_params=pltpu.CompilerParams(use_tc_tiling_on_sc=True, needs_layout_passes=False),
    )(gidx, offs, table4)
    return out.reshape(NSEQ, SEQ_LEN, EMBED_DIM)


def kernel(word_sequences, table):
    return _embed_gather(word_sequences, table)


# speedup vs baseline: 1.6006x; 1.6006x over previous
import jax
import jax.numpy as jnp
from jax import lax
from jax.experimental import pallas as pl
from jax.experimental.pallas import tpu as pltpu
from jax.experimental.pallas import tpu_sc as plsc

VOCAB_SIZE = 1000000
EMBED_DIM = 32

NUM_CORES = 2
NUM_SUBCORES = 16
NUM_WORKERS = NUM_CORES * NUM_SUBCORES

NSEQ = 4096
SEQ_LEN = 200
LANES = 128
ROWS_PER_LINE = LANES // EMBED_DIM
N_LINES = VOCAB_SIZE // ROWS_PER_LINE

S_PER_CHUNK = 4
N_CHUNKS = SEQ_LEN // S_PER_CHUNK
GROUPS = LANES // 16


def _gather_body(gidx_hbm, off_hbm, table_hbm, out_hbm,
                 gidx_v, slab0, slab1, slab2, slab3, ostage,
                 ob0, ob1, ob2,
                 g0, g1, g2, g3, b0, b1, b2):
    slabs = [slab0, slab1, slab2, slab3]
    gsems = [g0, g1, g2, g3]
    oblks = [ob0, ob1, ob2]
    bsems = [b0, b1, b2]

    wid = lax.axis_index("s") * NUM_CORES + lax.axis_index("c")
    lane0 = wid * LANES
    pltpu.sync_copy(gidx_hbm.at[:, pl.ds(lane0, LANES)], gidx_v)

    iota = lax.iota(jnp.int32, 16)
    rowmul = [(iota + 16 * g) * LANES for g in range(GROUPS)]

    def fire_stream(s, b):
        pltpu.async_copy(table_hbm.at[gidx_v.at[s]], slabs[b], gsems[b])

    def drain_stream(b):
        pltpu.make_async_copy(
            table_hbm.at[pl.ds(0, LANES)], slabs[b], gsems[b]
        ).wait()

    def stage_off(c, k):
        c = jnp.minimum(c, N_CHUNKS - 1)
        pltpu.async_copy(
            off_hbm.at[pl.ds(c * S_PER_CHUNK, S_PER_CHUNK),
                       pl.ds(lane0, LANES)],
            oblks[k], bsems[k],
        )

    def drain_off(k):
        pltpu.make_async_copy(
            off_hbm.at[pl.ds(0, S_PER_CHUNK), pl.ds(lane0, LANES)],
            oblks[k], bsems[k],
        ).wait()

    def extract(s4, sb, k):
        slab = slabs[sb]
        offs = [oblks[k][s4, pl.ds(16 * g, 16)] for g in range(GROUPS)]

        def fbody(f, carry):
            for g in range(GROUPS):
                v = plsc.load_gather(slab, [iota + 16 * g, offs[g] + f])
                ostage[s4, f, pl.ds(16 * g, 16)] = v
            return carry

        lax.fori_loop(0, EMBED_DIM, fbody, 0)

    def chunk(c, k, fire_next):
        drain_off(k)
        for s4 in range(S_PER_CHUNK):
            drain_stream(s4)
            extract(s4, s4, k)
            if fire_next:
                fire_stream((c + 1) * S_PER_CHUNK + s4, s4)
        pltpu.sync_copy(
            ostage,
            out_hbm.at[pl.ds(c * S_PER_CHUNK, S_PER_CHUNK), :,
                       pl.ds(lane0, LANES)],
        )
        stage_off(c + 2, (k + 2) % 3)

    stage_off(0, 0)
    stage_off(1, 1)
    for s4 in range(S_PER_CHUNK):
        fire_stream(s4, s4)
    chunk(0, 0, True)

    def main_body(j, carry):
        c0 = 3 * j + 1
        for t in range(3):
            chunk(c0 + t, (1 + t) % 3, True)
        return carry

    lax.fori_loop(0, (N_CHUNKS - 2) // 3, main_body, 0)

    chunk(N_CHUNKS - 1, (N_CHUNKS - 1) % 3, False)


@jax.jit
def _embed_gather(word_sequences, table):
    idx = word_sequences.astype(jnp.int32)
    gidx_t = (idx >> 2).T
    off_t = ((idx & 3) << 5).T
    table4 = table.reshape(N_LINES, LANES)
    mesh = plsc.VectorSubcoreMesh(
        core_axis_name="c",
        subcore_axis_name="s",
        num_cores=NUM_CORES,
        num_subcores=NUM_SUBCORES,
    )
    out = pl.kernel(
        _gather_body,
        out_type=jax.ShapeDtypeStruct((SEQ_LEN, EMBED_DIM, NSEQ), jnp.float32),
        mesh=mesh,
        scratch_types=(
            [pltpu.VMEM((SEQ_LEN, LANES), jnp.int32)]
            + [pltpu.VMEM((LANES, LANES), jnp.float32) for _ in range(4)]
            + [pltpu.VMEM((S_PER_CHUNK, EMBED_DIM, LANES), jnp.float32)]
            + [pltpu.VMEM((S_PER_CHUNK, LANES), jnp.int32) for _ in range(3)]
            + [pltpu.SemaphoreType.DMA for _ in range(7)]
        ),
        compiler_params=pltpu.CompilerParams(
            use_tc_tiling_on_sc=True, needs_layout_passes=False
        ),
    )(gidx_t, off_t, table4)
    return jnp.transpose(out, (2, 0, 1))


def kernel(word_sequences, table):
    return _embed_gather(word_sequences, table)
